# baseline (speedup 1.0000x reference)
import jax
import jax.numpy as jnp
from jax import lax
from jax.experimental import pallas as pl
from jax.experimental.pallas import tpu as pltpu
from jax.experimental.pallas import tpu_sc as plsc

N_NODES = 10000
D_FEAT = 128
HALF = 64
N_EDGES = 320000

N_PAD = 10240
RPT = N_PAD // 16
CH = 128
NCH = 160
E_TILE = NCH * CH
E_PAD = 16 * E_TILE
DUMMY = 10100
NSUB = RPT // CH


def _normalize_body(f_ref, o_ref):
    x = f_ref[...]
    n2 = jnp.sum(x * x, axis=1, keepdims=True)
    nrm = jnp.sqrt(n2)
    o_ref[...] = x / jnp.maximum(nrm, 1e-12)


def _tc_normalize(features):
    return pl.pallas_call(
        _normalize_body,
        grid=(10,),
        in_specs=[pl.BlockSpec((1000, 128), lambda i: (i, 0))],
        out_specs=pl.BlockSpec((1000, 128), lambda i: (i, 0)),
        out_shape=jax.ShapeDtypeStruct((N_NODES, D_FEAT), jnp.float32),
    )(features)


def _rsqrt16(v):
    i = lax.bitcast_convert_type(v, jnp.int32)
    y = lax.bitcast_convert_type(jnp.int32(0x5F3759DF) - (i >> 1), jnp.float32)
    for _ in range(2):
        y = y * (1.5 - 0.5 * v * y * y)
    return y


def _sc_body(row_hbm, col_hbm, x_hbm, out_hbm, h_hbm, t_hbm,
             dis_v, idx_v, ones_v, zb_v, g0, g1, g2, g3,
             rb0, rb1, rb2, rb3, rb4, rb5, rb6, rb7,
             cb0, cb1, cb2, cb3, cb4, cb5, cb6, cb7,
             xs_sh, a_sh, deg_sh,
             sem, si0, si1, si2, si3, sg0, sg1, sg2, sg3,
             ss0, ss1, ss2, ss3):
    cid = lax.axis_index("c")
    sid = lax.axis_index("s")
    base_r = sid * RPT
    coff = cid * N_PAD
    hb = pl.multiple_of(coff + base_r, CH)

    z16 = jnp.zeros((16,), jnp.float32)
    one16 = jnp.ones((16,), jnp.float32)
    dummy16 = jnp.full((16,), DUMMY, jnp.int32)
    gs = (g0, g1, g2, g3)
    sgs = (sg0, sg1, sg2, sg3)
    sss = (ss0, ss1, ss2, ss3)
    sis = (si0, si1, si2, si3)
    rbs = (rb0, rb1, rb2, rb3, rb4, rb5, rb6, rb7)
    cbs = (cb0, cb1, cb2, cb3, cb4, cb5, cb6, cb7)

    def idx_issue(jj, s8, s4):
        pltpu.async_copy(row_hbm.at[sid, jj], rbs[s8], sis[s4])
        pltpu.async_copy(col_hbm.at[sid, jj], cbs[s8], sis[s4])

    def idx_wait(jj, s8, s4):
        pltpu.make_async_copy(row_hbm.at[sid, jj], rbs[s8], sis[s4]).wait()
        pltpu.make_async_copy(col_hbm.at[sid, jj], cbs[s8], sis[s4]).wait()

    HSET = (0, 3, 6)

    def redirect(s8, hbm=False):
        for k in range(8):
            sl = pl.ds(16 * k, 16)
            r = rbs[s8][sl]
            c = cbs[s8][sl]
            rd = jnp.where(r != c, r, dummy16)
            if hbm:
                rd = rd + coff
            rbs[s8][sl] = rd

    def zrow(i, _):
        for k in range(4):
            g0[i, pl.ds(16 * k, 16)] = z16
        return 0
    lax.fori_loop(0, CH, zrow, 0)

    def zzb(i, _):
        zb_v[pl.ds(i * 16, 16)] = z16
        return 0
    lax.fori_loop(0, RPT // 16, zzb, 0)
    for k in range(8):
        ones_v[pl.ds(16 * k, 16)] = one16

    icps = [pltpu.async_copy(g0, a_sh.at[pl.ds(base_r + CH * s, CH)], sem)
            for s in range(NSUB)]
    icps.append(pltpu.async_copy(zb_v, deg_sh.at[pl.ds(base_r, RPT)], sem))
    for c in icps:
        c.wait()
    plsc.subcore_barrier()

    for c in range(4):
        idx_issue(c, c, c)

    def degstep(j, s8, s4):
        idx_wait(j, s8, s4)
        redirect(s8)

        @pl.when(j >= 4)
        def _():
            pltpu.make_async_copy(ones_v, deg_sh.at[rbs[(s8 + 4) % 8]],
                                  sss[s4]).wait()
        pltpu.async_copy(ones_v, deg_sh.at[rbs[s8]], sss[s4], add=True)

        @pl.when(j + 4 < NCH)
        def _():
            idx_issue(j + 4, (s8 + 4) % 8, s4)

    def deggroup(i, _):
        for u in range(8):
            degstep(8 * i + u, u, u % 4)
        return 0
    lax.fori_loop(0, NCH // 8, deggroup, 0)
    for j in range(NCH - 4, NCH):
        pltpu.make_async_copy(ones_v, deg_sh.at[rbs[j % 8]],
                              sss[j % 4]).wait()
    plsc.subcore_barrier()

    pltpu.sync_copy(deg_sh, dis_v)

    def disrow(i, _):
        sl = pl.ds(16 * i, 16)
        dis_v[sl] = _rsqrt16(dis_v[sl])
        return 0
    lax.fori_loop(0, N_PAD // 16, disrow, 0)

    def idxrow(i, _):
        lane = lax.iota(jnp.int32, 16) + (base_r + 16 * i)
        idx_v[pl.ds(16 * i, 16)] = lane * 2 + cid
        return 0
    lax.fori_loop(0, RPT // 16, idxrow, 0)

    def rowscale(dst, src, b0):
        def grp(g, _):
            dv = dis_v[pl.ds(b0 + 16 * g, 16)]
            for t in range(16):
                sv = dv[t]
                for k in range(4):
                    sl = pl.ds(16 * k, 16)
                    dst[16 * g + t, sl] = src[16 * g + t, sl] * sv
            return 0
        lax.fori_loop(0, 8, grp, 0)

    wcps = []
    for s in range(NSUB):
        b0 = base_r + CH * s
        ga, gb = gs[2 * (s % 2)], gs[2 * (s % 2) + 1]
        if s >= 2:
            wcps[s - 2].wait()
        pltpu.async_copy(x_hbm.at[idx_v.at[pl.ds(CH * s, CH)]], ga,
                         sgs[s % 2]).wait()
        rowscale(gb, ga, b0)
        wcps.append(pltpu.async_copy(gb, t_hbm.at[pl.ds(hb + CH * s, CH)],
                                     sss[s % 2]))
        pltpu.sync_copy(gb, xs_sh.at[pl.ds(b0, CH)])
    for c in wcps[NSUB - 2:]:
        c.wait()
    plsc.subcore_barrier()

    def conv(src_sh, acc_sh):
        def src_ref(s8):
            return t_hbm if s8 in HSET else src_sh

        def gather_issue(jj, s8, s4):
            pltpu.async_copy(src_ref(s8).at[rbs[s8]], gs[s4], sgs[s4])

        def gather_wait(jj, s8, s4):
            pltpu.make_async_copy(src_ref(s8).at[rbs[s8]],
                                  gs[s4], sgs[s4]).wait()

        def scat_issue(jj, s8, s4):
            pltpu.async_copy(gs[s4], acc_sh.at[cbs[s8]], sss[s4], add=True)

        def scat_wait(jj, s8, s4):
            pltpu.make_async_copy(gs[s4], acc_sh.at[cbs[s8]], sss[s4]).wait()

        for c in range(4):
            idx_issue(c, c, c)
        for c in range(2):
            idx_wait(c, c, c)
            redirect(c, hbm=c in HSET)
            gather_issue(c, c, c)
            idx_issue(c + 4, c + 4, c)

        for j in (0, 1):
            idx_wait(j + 2, j + 2, j + 2)
            redirect(j + 2, hbm=(j + 2) in HSET)
            gather_issue(j + 2, j + 2, j + 2)
            idx_issue(j + 6, j + 6, j + 2)
            gather_wait(j, j, j)
            scat_issue(j, j, j)

        def step(j, s8, s4):
            scat_wait(j - 2, (s8 + 6) % 8, (s4 + 2) % 4)
            idx_wait(j + 2, (s8 + 2) % 8, (s4 + 2) % 4)
            redirect((s8 + 2) % 8, hbm=(s8 + 2) % 8 in HSET)
            gather_issue(j + 2, (s8 + 2) % 8, (s4 + 2) % 4)
            idx_issue(j + 6, (s8 + 6) % 8, (s4 + 2) % 4)
            gather_wait(j, s8, s4)
            scat_issue(j, s8, s4)

        def group(i, _):
            for u in range(8):
                j = 8 * i + u + 2
                step(j, (u + 2) % 8, (u + 2) % 4)
            return 0
        lax.fori_loop(0, (NCH - 8) // 8, group, 0)

        for j in range(NCH - 6, NCH):
            scat_wait(j - 2, (j - 2) % 8, (j - 2) % 4)
            if j + 2 < NCH:
                idx_wait(j + 2, (j + 2) % 8, (j + 2) % 4)
                redirect((j + 2) % 8, hbm=(j + 2) % 8 in HSET)
                gather_issue(j + 2, (j + 2) % 8, (j + 2) % 4)
            gather_wait(j, j % 8, j % 4)
            scat_issue(j, j % 8, j % 4)
        for j in range(NCH - 2, NCH):
            scat_wait(j, j % 8, j % 4)

    conv(xs_sh, a_sh)
    plsc.subcore_barrier()

    tcps = []
    for s in range(NSUB):
        b0 = base_r + CH * s
        ga, gb = gs[2 * (s % 2)], gs[2 * (s % 2) + 1]
        if s >= 2:
            tcps[2 * (s - 2)].wait()
            tcps[2 * (s - 2) + 1].wait()
        pltpu.sync_copy(a_sh.at[pl.ds(b0, CH)], ga)
        rowscale(ga, ga, b0)
        tcps.append(pltpu.async_copy(ga, h_hbm.at[pl.ds(hb + CH * s, CH)],
                                     sgs[s % 2]))
        rowscale(gb, ga, b0)
        tcps.append(pltpu.async_copy(gb, t_hbm.at[pl.ds(hb + CH * s, CH)],
                                     sss[s % 2]))
        pltpu.sync_copy(gb, xs_sh.at[pl.ds(b0, CH)])
    for c in tcps[2 * (NSUB - 2):]:
        c.wait()

    def zrow2(i, _):
        for k in range(4):
            g0[i, pl.ds(16 * k, 16)] = z16
        return 0
    lax.fori_loop(0, CH, zrow2, 0)
    zcps = [pltpu.async_copy(g0, a_sh.at[pl.ds(base_r + CH * s, CH)], sem)
            for s in range(NSUB)]
    for c in zcps:
        c.wait()
    plsc.subcore_barrier()

    conv(xs_sh, a_sh)
    plsc.subcore_barrier()

    for s in range(NSUB):
        b0 = base_r + CH * s
        cx = pltpu.async_copy(x_hbm.at[idx_v.at[pl.ds(CH * s, CH)]], g0, sg0)
        pltpu.sync_copy(a_sh.at[pl.ds(b0, CH)], g1)
        rowscale(g1, g1, b0)
        pltpu.sync_copy(h_hbm.at[pl.ds(hb + CH * s, CH)], g2)
        cx.wait()

        def addrow(i, _):
            for k in range(4):
                sl = pl.ds(16 * k, 16)
                g0[i, sl] = g0[i, sl] + g1[i, sl] + g2[i, sl]
            return 0
        lax.fori_loop(0, CH, addrow, 0)
        pltpu.sync_copy(g0, out_hbm.at[cid, pl.ds(b0, CH)])


def _sc_gcn(row_t, col_t, xflat):
    mesh = plsc.VectorSubcoreMesh(core_axis_name="c", subcore_axis_name="s")
    return pl.kernel(
        _sc_body,
        out_type=[
            jax.ShapeDtypeStruct((2, N_PAD, HALF), jnp.float32),
            jax.ShapeDtypeStruct((2 * N_PAD, HALF), jnp.float32),
            jax.ShapeDtypeStruct((2 * N_PAD, HALF), jnp.float32),
        ],
        mesh=mesh,
        compiler_params=pltpu.CompilerParams(needs_layout_passes=False,
                                             use_tc_tiling_on_sc=False),
        scratch_types=[
            pltpu.VMEM((N_PAD,), jnp.float32),
            pltpu.VMEM((RPT,), jnp.int32),
            pltpu.VMEM((CH,), jnp.float32),
            pltpu.VMEM((RPT,), jnp.float32),
            pltpu.VMEM((CH, HALF), jnp.float32),
            pltpu.VMEM((CH, HALF), jnp.float32),
            pltpu.VMEM((CH, HALF), jnp.float32),
            pltpu.VMEM((CH, HALF), jnp.float32),
        ] + [pltpu.VMEM((CH,), jnp.int32) for _ in range(16)]
        + [
            pltpu.VMEM_SHARED((N_PAD, HALF), jnp.float32),
            pltpu.VMEM_SHARED((N_PAD, HALF), jnp.float32),
            pltpu.VMEM_SHARED((N_PAD,), jnp.float32),
        ] + [pltpu.SemaphoreType.DMA] * 13,
    )(row_t, col_t, xflat)


def kernel(edge_index_drop, edge_index, features, preference):
    del edge_index_drop
    x = _tc_normalize(features.astype(jnp.float32))

    xpad = jnp.pad(x, ((0, N_PAD - N_NODES), (0, 0)))
    xflat = xpad.reshape(2 * N_PAD, HALF)

    ei = edge_index.astype(jnp.int32)
    rowp = jnp.pad(ei[0], (0, E_PAD - N_EDGES)).reshape(16, NCH, CH)
    colp = jnp.pad(ei[1], (0, E_PAD - N_EDGES)).reshape(16, NCH, CH)

    out_split, _h, _t = _sc_gcn(rowp, colp, xflat)
    x_hat = out_split.transpose(1, 0, 2).reshape(N_PAD, D_FEAT)[:N_NODES]
    return (x_hat, preference)


# speedup vs baseline: 1.2819x; 1.2819x over previous
import jax
import jax.numpy as jnp
from jax import lax
from jax.experimental import pallas as pl
from jax.experimental.pallas import tpu as pltpu
from jax.experimental.pallas import tpu_sc as plsc

N_NODES = 10000
D_FEAT = 128
HALF = 64
N_EDGES = 320000

N_PAD = 10240
RPT = N_PAD // 16
CH = 128
NCH = 160
E_TILE = NCH * CH
E_PAD = 16 * E_TILE
DUMMY = 10100
NSUB = RPT // CH


def _normalize_body(f_ref, o_ref):
    x = f_ref[...]
    n2 = jnp.sum(x * x, axis=1, keepdims=True)
    nrm = jnp.sqrt(n2)
    o_ref[...] = x / jnp.maximum(nrm, 1e-12)


def _tc_normalize(features):
    return pl.pallas_call(
        _normalize_body,
        grid=(10,),
        in_specs=[pl.BlockSpec((1000, 128), lambda i: (i, 0))],
        out_specs=pl.BlockSpec((1000, 128), lambda i: (i, 0)),
        out_shape=jax.ShapeDtypeStruct((N_NODES, D_FEAT), jnp.float32),
    )(features)


def _rsqrt16(v):
    i = lax.bitcast_convert_type(v, jnp.int32)
    y = lax.bitcast_convert_type(jnp.int32(0x5F3759DF) - (i >> 1), jnp.float32)
    for _ in range(2):
        y = y * (1.5 - 0.5 * v * y * y)
    return y


def _sc_body(row_hbm, col_hbm, x_hbm, out_hbm, h_hbm,
             dis_v, idx_v, ones_v, zb_v, g0, g1, g2, g3,
             rb0, rb1, rb2, rb3, rb4, rb5, rb6, rb7,
             cb0, cb1, cb2, cb3, cb4, cb5, cb6, cb7,
             xs_sh, a_sh, deg_sh,
             sem, si0, si1, si2, si3, sg0, sg1, sg2, sg3,
             ss0, ss1, ss2, ss3):
    cid = lax.axis_index("c")
    sid = lax.axis_index("s")
    base_r = sid * RPT
    hb = pl.multiple_of(cid * N_PAD + base_r, CH)

    z16 = jnp.zeros((16,), jnp.float32)
    one16 = jnp.ones((16,), jnp.float32)
    dummy16 = jnp.full((16,), DUMMY, jnp.int32)
    gs = (g0, g1, g2, g3)
    sgs = (sg0, sg1, sg2, sg3)
    sss = (ss0, ss1, ss2, ss3)
    sis = (si0, si1, si2, si3)
    rbs = (rb0, rb1, rb2, rb3, rb4, rb5, rb6, rb7)
    cbs = (cb0, cb1, cb2, cb3, cb4, cb5, cb6, cb7)

    def idx_issue(jj, s8, s4):
        pltpu.async_copy(row_hbm.at[sid, jj], rbs[s8], sis[s4])
        pltpu.async_copy(col_hbm.at[sid, jj], cbs[s8], sis[s4])

    def idx_wait(jj, s8, s4):
        pltpu.make_async_copy(row_hbm.at[sid, jj], rbs[s8], sis[s4]).wait()
        pltpu.make_async_copy(col_hbm.at[sid, jj], cbs[s8], sis[s4]).wait()

    def redirect(s8):
        for k in range(8):
            sl = pl.ds(16 * k, 16)
            r = rbs[s8][sl]
            c = cbs[s8][sl]
            rbs[s8][sl] = jnp.where(r != c, r, dummy16)

    def zrow(i, _):
        for k in range(4):
            g0[i, pl.ds(16 * k, 16)] = z16
        return 0
    lax.fori_loop(0, CH, zrow, 0)

    def zzb(i, _):
        zb_v[pl.ds(i * 16, 16)] = z16
        return 0
    lax.fori_loop(0, RPT // 16, zzb, 0)
    for k in range(8):
        ones_v[pl.ds(16 * k, 16)] = one16

    icps = [pltpu.async_copy(g0, a_sh.at[pl.ds(base_r + CH * s, CH)], sem)
            for s in range(NSUB)]
    icps.append(pltpu.async_copy(zb_v, deg_sh.at[pl.ds(base_r, RPT)], sem))
    for c in icps:
        c.wait()
    plsc.subcore_barrier()

    for c in range(4):
        idx_issue(c, c, c)

    def degstep(j, s8, s4):
        idx_wait(j, s8, s4)
        redirect(s8)

        @pl.when(j >= 4)
        def _():
            pltpu.make_async_copy(ones_v, deg_sh.at[rbs[(s8 + 4) % 8]],
                                  sss[s4]).wait()
        pltpu.async_copy(ones_v, deg_sh.at[rbs[s8]], sss[s4], add=True)

        @pl.when(j + 4 < NCH)
        def _():
            idx_issue(j + 4, (s8 + 4) % 8, s4)

    def deggroup(i, _):
        for u in range(8):
            degstep(8 * i + u, u, u % 4)
        return 0
    lax.fori_loop(0, NCH // 8, deggroup, 0)
    for j in range(NCH - 4, NCH):
        pltpu.make_async_copy(ones_v, deg_sh.at[rbs[j % 8]],
                              sss[j % 4]).wait()
    plsc.subcore_barrier()

    pltpu.sync_copy(deg_sh, dis_v)

    def disrow(i, _):
        sl = pl.ds(16 * i, 16)
        dis_v[sl] = _rsqrt16(dis_v[sl])
        return 0
    lax.fori_loop(0, N_PAD // 16, disrow, 0)

    def idxrow(i, _):
        lane = lax.iota(jnp.int32, 16) + (base_r + 16 * i)
        idx_v[pl.ds(16 * i, 16)] = lane * 2 + cid
        return 0
    lax.fori_loop(0, RPT // 16, idxrow, 0)

    def rowscale(dst, src, b0):
        def grp(g, _):
            dv = dis_v[pl.ds(b0 + 16 * g, 16)]
            for t in range(16):
                sv = dv[t]
                for k in range(4):
                    sl = pl.ds(16 * k, 16)
                    dst[16 * g + t, sl] = src[16 * g + t, sl] * sv
            return 0
        lax.fori_loop(0, 8, grp, 0)

    for s in range(NSUB):
        b0 = base_r + CH * s
        ga, gb = gs[2 * (s % 2)], gs[2 * (s % 2) + 1]
        pltpu.async_copy(x_hbm.at[idx_v.at[pl.ds(CH * s, CH)]], ga,
                         sgs[s % 2]).wait()
        rowscale(gb, ga, b0)
        pltpu.sync_copy(gb, xs_sh.at[pl.ds(b0, CH)])
    plsc.subcore_barrier()

    def conv(src_sh, acc_sh):
        def gather_issue(jj, s8, s4):
            pltpu.async_copy(src_sh.at[rbs[s8]], gs[s4], sgs[s4])

        def gather_wait(jj, s8, s4):
            pltpu.make_async_copy(src_sh.at[rbs[s8]], gs[s4], sgs[s4]).wait()

        def scat_issue(jj, s8, s4):
            pltpu.async_copy(gs[s4], acc_sh.at[cbs[s8]], sss[s4], add=True)

        def scat_wait(jj, s8, s4):
            pltpu.make_async_copy(gs[s4], acc_sh.at[cbs[s8]], sss[s4]).wait()

        for c in range(4):
            idx_issue(c, c, c)
        for c in range(2):
            idx_wait(c, c, c)
            redirect(c)
            gather_issue(c, c, c)
            idx_issue(c + 4, c + 4, c)

        for j in (0, 1):
            idx_wait(j + 2, j + 2, j + 2)
            redirect(j + 2)
            gather_issue(j + 2, j + 2, j + 2)
            idx_issue(j + 6, j + 6, j + 2)
            gather_wait(j, j, j)
            scat_issue(j, j, j)

        def step(j, s8, s4):
            scat_wait(j - 2, (s8 + 6) % 8, (s4 + 2) % 4)
            idx_wait(j + 2, (s8 + 2) % 8, (s4 + 2) % 4)
            redirect((s8 + 2) % 8)
            gather_issue(j + 2, (s8 + 2) % 8, (s4 + 2) % 4)
            idx_issue(j + 6, (s8 + 6) % 8, (s4 + 2) % 4)
            gather_wait(j, s8, s4)
            scat_issue(j, s8, s4)

        def group(i, _):
            for u in range(8):
                j = 8 * i + u + 2
                step(j, (u + 2) % 8, (u + 2) % 4)
            return 0
        lax.fori_loop(0, (NCH - 8) // 8, group, 0)

        for j in range(NCH - 6, NCH):
            scat_wait(j - 2, (j - 2) % 8, (j - 2) % 4)
            if j + 2 < NCH:
                idx_wait(j + 2, (j + 2) % 8, (j + 2) % 4)
                redirect((j + 2) % 8)
                gather_issue(j + 2, (j + 2) % 8, (j + 2) % 4)
            gather_wait(j, j % 8, j % 4)
            scat_issue(j, j % 8, j % 4)
        for j in range(NCH - 2, NCH):
            scat_wait(j, j % 8, j % 4)

    conv(xs_sh, a_sh)
    plsc.subcore_barrier()

    tcps = []
    for s in range(NSUB):
        b0 = base_r + CH * s
        ga, gb = gs[2 * (s % 2)], gs[2 * (s % 2) + 1]
        if s >= 2:
            tcps[s - 2].wait()
        pltpu.sync_copy(a_sh.at[pl.ds(b0, CH)], ga)
        rowscale(ga, ga, b0)
        tcps.append(pltpu.async_copy(ga, h_hbm.at[pl.ds(hb + CH * s, CH)],
                                     sgs[s % 2]))
        rowscale(gb, ga, b0)
        pltpu.sync_copy(gb, xs_sh.at[pl.ds(b0, CH)])
    for c in tcps[NSUB - 2:]:
        c.wait()

    def zrow2(i, _):
        for k in range(4):
            g0[i, pl.ds(16 * k, 16)] = z16
        return 0
    lax.fori_loop(0, CH, zrow2, 0)
    zcps = [pltpu.async_copy(g0, a_sh.at[pl.ds(base_r + CH * s, CH)], sem)
            for s in range(NSUB)]
    for c in zcps:
        c.wait()
    plsc.subcore_barrier()

    conv(xs_sh, a_sh)
    plsc.subcore_barrier()

    for s in range(NSUB):
        b0 = base_r + CH * s
        cx = pltpu.async_copy(x_hbm.at[idx_v.at[pl.ds(CH * s, CH)]], g0, sg0)
        pltpu.sync_copy(a_sh.at[pl.ds(b0, CH)], g1)
        rowscale(g1, g1, b0)
        pltpu.sync_copy(h_hbm.at[pl.ds(hb + CH * s, CH)], g2)
        cx.wait()

        def addrow(i, _):
            for k in range(4):
                sl = pl.ds(16 * k, 16)
                g0[i, sl] = g0[i, sl] + g1[i, sl] + g2[i, sl]
            return 0
        lax.fori_loop(0, CH, addrow, 0)
        pltpu.sync_copy(g0, out_hbm.at[pl.ds(b0, CH), cid])


def _sc_gcn(row_t, col_t, xflat):
    mesh = plsc.VectorSubcoreMesh(core_axis_name="c", subcore_axis_name="s")
    return pl.kernel(
        _sc_body,
        out_type=[
            jax.ShapeDtypeStruct((N_PAD, 2, HALF), jnp.float32),
            jax.ShapeDtypeStruct((2 * N_PAD, HALF), jnp.float32),
        ],
        mesh=mesh,
        compiler_params=pltpu.CompilerParams(needs_layout_passes=False,
                                             use_tc_tiling_on_sc=False),
        scratch_types=[
            pltpu.VMEM((N_PAD,), jnp.float32),
            pltpu.VMEM((RPT,), jnp.int32),
            pltpu.VMEM((CH,), jnp.float32),
            pltpu.VMEM((RPT,), jnp.float32),
            pltpu.VMEM((CH, HALF), jnp.float32),
            pltpu.VMEM((CH, HALF), jnp.float32),
            pltpu.VMEM((CH, HALF), jnp.float32),
            pltpu.VMEM((CH, HALF), jnp.float32),
        ] + [pltpu.VMEM((CH,), jnp.int32) for _ in range(16)]
        + [
            pltpu.VMEM_SHARED((N_PAD, HALF), jnp.float32),
            pltpu.VMEM_SHARED((N_PAD, HALF), jnp.float32),
            pltpu.VMEM_SHARED((N_PAD,), jnp.float32),
        ] + [pltpu.SemaphoreType.DMA] * 13,
    )(row_t, col_t, xflat)


def kernel(edge_index_drop, edge_index, features, preference):
    del edge_index_drop
    x = _tc_normalize(features.astype(jnp.float32))

    xpad = jnp.pad(x, ((0, N_PAD - N_NODES), (0, 0)))
    xflat = xpad.reshape(2 * N_PAD, HALF)

    ei = edge_index.astype(jnp.int32)
    rowp = jnp.pad(ei[0], (0, E_PAD - N_EDGES)).reshape(16, NCH, CH)
    colp = jnp.pad(ei[1], (0, E_PAD - N_EDGES)).reshape(16, NCH, CH)

    out_split, _h = _sc_gcn(rowp, colp, xflat)
    x_hat = out_split.reshape(N_PAD, D_FEAT)[:N_NODES]
    return (x_hat, preference)


# speedup vs baseline: 1.3549x; 1.0569x over previous
import jax
import jax.numpy as jnp
from jax import lax
from jax.experimental import pallas as pl
from jax.experimental.pallas import tpu as pltpu
from jax.experimental.pallas import tpu_sc as plsc

N_NODES = 10000
D_FEAT = 128
HALF = 64
N_EDGES = 320000

N_PAD = 10240
RPT = N_PAD // 16
CH = 128
NCH = 160
E_TILE = NCH * CH
E_PAD = 16 * E_TILE
DUMMY = 10100
NSUB = RPT // CH


def _normalize_body(f_ref, o_ref):
    x = f_ref[...]
    n2 = jnp.sum(x * x, axis=1, keepdims=True)
    nrm = jnp.sqrt(n2)
    o_ref[...] = x / jnp.maximum(nrm, 1e-12)


def _tc_normalize(features):
    return pl.pallas_call(
        _normalize_body,
        grid=(10,),
        in_specs=[pl.BlockSpec((1000, 128), lambda i: (i, 0))],
        out_specs=pl.BlockSpec((1000, 128), lambda i: (i, 0)),
        out_shape=jax.ShapeDtypeStruct((N_NODES, D_FEAT), jnp.float32),
    )(features)


def _rsqrt16(v):
    i = lax.bitcast_convert_type(v, jnp.int32)
    y = lax.bitcast_convert_type(jnp.int32(0x5F3759DF) - (i >> 1), jnp.float32)
    for _ in range(2):
        y = y * (1.5 - 0.5 * v * y * y)
    return y


def _sc_body(row_hbm, col_hbm, x_hbm, out_hbm, h_hbm,
             dis_v, idx_v, ones_v, zb_v, g0, g1, g2, g3,
             rb0, rb1, rb2, rb3, rb4, rb5, rb6, rb7,
             cb0, cb1, cb2, cb3, cb4, cb5, cb6, cb7,
             xs_sh, a_sh, deg_sh,
             sem, si0, si1, si2, si3, sg0, sg1, sg2, sg3,
             ss0, ss1, ss2, ss3):
    cid = lax.axis_index("c")
    sid = lax.axis_index("s")
    base_r = sid * RPT
    hb = pl.multiple_of(cid * N_PAD + base_r, CH)

    z16 = jnp.zeros((16,), jnp.float32)
    one16 = jnp.ones((16,), jnp.float32)
    dummy16 = jnp.full((16,), DUMMY, jnp.int32)
    gs = (g0, g1, g2, g3)
    sgs = (sg0, sg1, sg2, sg3)
    sss = (ss0, ss1, ss2, ss3)
    sis = (si0, si1, si2, si3)
    rbs = (rb0, rb1, rb2, rb3, rb4, rb5, rb6, rb7)
    cbs = (cb0, cb1, cb2, cb3, cb4, cb5, cb6, cb7)

    def idx_issue(jj, s8, s4):
        pltpu.async_copy(row_hbm.at[sid, jj], rbs[s8], sis[s4])
        pltpu.async_copy(col_hbm.at[sid, jj], cbs[s8], sis[s4])

    def idx_wait(jj, s8, s4):
        pltpu.make_async_copy(row_hbm.at[sid, jj], rbs[s8], sis[s4]).wait()
        pltpu.make_async_copy(col_hbm.at[sid, jj], cbs[s8], sis[s4]).wait()

    def redirect(s8):
        for k in range(8):
            sl = pl.ds(16 * k, 16)
            r = rbs[s8][sl]
            c = cbs[s8][sl]
            rbs[s8][sl] = jnp.where(r != c, r, dummy16)

    def zrow(i, _):
        for k in range(4):
            g0[i, pl.ds(16 * k, 16)] = z16
        return 0
    lax.fori_loop(0, CH, zrow, 0)

    def zzb(i, _):
        zb_v[pl.ds(i * 16, 16)] = z16
        return 0
    lax.fori_loop(0, RPT // 16, zzb, 0)
    for k in range(8):
        ones_v[pl.ds(16 * k, 16)] = one16

    icps = [pltpu.async_copy(g0, a_sh.at[pl.ds(base_r + CH * s, CH)], sem)
            for s in range(NSUB)]
    icps.append(pltpu.async_copy(zb_v, deg_sh.at[pl.ds(base_r, RPT)], sem))
    for c in icps:
        c.wait()
    plsc.subcore_barrier()

    for c in range(4):
        idx_issue(c, c, c)

    def degstep(j, s8, s4):
        idx_wait(j, s8, s4)
        redirect(s8)

        @pl.when(j >= 4)
        def _():
            pltpu.make_async_copy(ones_v, deg_sh.at[rbs[(s8 + 4) % 8]],
                                  sss[s4]).wait()
        pltpu.async_copy(ones_v, deg_sh.at[rbs[s8]], sss[s4], add=True)

        @pl.when(j + 4 < NCH)
        def _():
            idx_issue(j + 4, (s8 + 4) % 8, s4)

    def deggroup(i, _):
        for u in range(8):
            degstep(8 * i + u, u, u % 4)
        return 0
    lax.fori_loop(0, NCH // 8, deggroup, 0)
    for j in range(NCH - 4, NCH):
        pltpu.make_async_copy(ones_v, deg_sh.at[rbs[j % 8]],
                              sss[j % 4]).wait()
    plsc.subcore_barrier()

    pltpu.sync_copy(deg_sh, dis_v)

    def disrow(i, _):
        sl = pl.ds(16 * i, 16)
        dis_v[sl] = _rsqrt16(dis_v[sl])
        return 0
    lax.fori_loop(0, N_PAD // 16, disrow, 0)

    def idxrow(i, _):
        lane = lax.iota(jnp.int32, 16) + (base_r + 16 * i)
        idx_v[pl.ds(16 * i, 16)] = lane * 2 + cid
        return 0
    lax.fori_loop(0, RPT // 16, idxrow, 0)

    def rowscale(dst, src, b0):
        def grp(g, _):
            dv = dis_v[pl.ds(b0 + 16 * g, 16)]
            for t in range(16):
                sv = dv[t]
                for k in range(4):
                    sl = pl.ds(16 * k, 16)
                    dst[16 * g + t, sl] = src[16 * g + t, sl] * sv
            return 0
        lax.fori_loop(0, 8, grp, 0)

    for s in range(NSUB):
        b0 = base_r + CH * s
        ga, gb = gs[2 * (s % 2)], gs[2 * (s % 2) + 1]
        pltpu.async_copy(x_hbm.at[idx_v.at[pl.ds(CH * s, CH)]], ga,
                         sgs[s % 2]).wait()
        rowscale(gb, ga, b0)
        pltpu.sync_copy(gb, xs_sh.at[pl.ds(b0, CH)])
    plsc.subcore_barrier()

    def conv(src_sh, acc_sh):
        def gather_issue(jj, s8, s4):
            pltpu.async_copy(src_sh.at[rbs[s8]], gs[s4], sgs[s4])

        def gather_wait(jj, s8, s4):
            pltpu.make_async_copy(src_sh.at[rbs[s8]], gs[s4], sgs[s4]).wait()

        def scat_issue(jj, s8, s4):
            pltpu.async_copy(gs[s4], acc_sh.at[cbs[s8]], sss[s4], add=True)

        def scat_wait(jj, s8, s4):
            pltpu.make_async_copy(gs[s4], acc_sh.at[cbs[s8]], sss[s4]).wait()

        for c in range(4):
            idx_issue(c, c, c)
        for c in range(2):
            idx_wait(c, c, c)
            redirect(c)
            gather_issue(c, c, c)
            idx_issue(c + 4, c + 4, c)

        for j in (0, 1):
            idx_wait(j + 2, j + 2, j + 2)
            redirect(j + 2)
            gather_issue(j + 2, j + 2, j + 2)
            idx_issue(j + 6, j + 6, j + 2)
            gather_wait(j, j, j)
            scat_issue(j, j, j)

        def step(j, s8, s4):
            scat_wait(j - 2, (s8 + 6) % 8, (s4 + 2) % 4)
            idx_wait(j + 2, (s8 + 2) % 8, (s4 + 2) % 4)
            redirect((s8 + 2) % 8)
            gather_issue(j + 2, (s8 + 2) % 8, (s4 + 2) % 4)
            idx_issue(j + 6, (s8 + 6) % 8, (s4 + 2) % 4)
            gather_wait(j, s8, s4)
            scat_issue(j, s8, s4)

        def group(i, _):
            for u in range(8):
                j = 8 * i + u + 2
                step(j, (u + 2) % 8, (u + 2) % 4)
            return 0
        lax.fori_loop(0, (NCH - 8) // 8, group, 0)

        for j in range(NCH - 6, NCH):
            scat_wait(j - 2, (j - 2) % 8, (j - 2) % 4)
            if j + 2 < NCH:
                idx_wait(j + 2, (j + 2) % 8, (j + 2) % 4)
                redirect((j + 2) % 8)
                gather_issue(j + 2, (j + 2) % 8, (j + 2) % 4)
            gather_wait(j, j % 8, j % 4)
            scat_issue(j, j % 8, j % 4)
        for j in range(NCH - 2, NCH):
            scat_wait(j, j % 8, j % 4)

    conv(xs_sh, a_sh)
    plsc.subcore_barrier()

    tcps = []
    for s in range(NSUB):
        b0 = base_r + CH * s
        ga, gb = gs[2 * (s % 2)], gs[2 * (s % 2) + 1]
        if s >= 2:
            tcps[s - 2].wait()
        pltpu.sync_copy(a_sh.at[pl.ds(b0, CH)], ga)
        rowscale(ga, ga, b0)
        tcps.append(pltpu.async_copy(ga, h_hbm.at[pl.ds(hb + CH * s, CH)],
                                     sgs[s % 2]))
        rowscale(gb, ga, b0)
        pltpu.sync_copy(gb, xs_sh.at[pl.ds(b0, CH)])
    for c in tcps[NSUB - 2:]:
        c.wait()

    def zrow2(i, _):
        for k in range(4):
            g0[i, pl.ds(16 * k, 16)] = z16
        return 0
    lax.fori_loop(0, CH, zrow2, 0)
    zcps = [pltpu.async_copy(g0, a_sh.at[pl.ds(base_r + CH * s, CH)], sem)
            for s in range(NSUB)]
    for c in zcps:
        c.wait()
    plsc.subcore_barrier()

    conv(xs_sh, a_sh)
    plsc.subcore_barrier()

    for s in range(NSUB):
        b0 = base_r + CH * s
        cx = pltpu.async_copy(x_hbm.at[idx_v.at[pl.ds(CH * s, CH)]], g0, sg0)
        pltpu.sync_copy(a_sh.at[pl.ds(b0, CH)], g1)
        rowscale(g1, g1, b0)
        pltpu.sync_copy(h_hbm.at[pl.ds(hb + CH * s, CH)], g2)
        cx.wait()

        def addrow(i, _):
            for k in range(4):
                sl = pl.ds(16 * k, 16)
                g0[i, sl] = g0[i, sl] + g1[i, sl] + g2[i, sl]
            return 0
        lax.fori_loop(0, CH, addrow, 0)
        pltpu.sync_copy(g0, out_hbm.at[cid, pl.ds(b0, CH)])


def _sc_gcn(row_t, col_t, xflat):
    mesh = plsc.VectorSubcoreMesh(core_axis_name="c", subcore_axis_name="s")
    return pl.kernel(
        _sc_body,
        out_type=[
            jax.ShapeDtypeStruct((2, N_PAD, HALF), jnp.float32),
            jax.ShapeDtypeStruct((2 * N_PAD, HALF), jnp.float32),
        ],
        mesh=mesh,
        compiler_params=pltpu.CompilerParams(needs_layout_passes=False,
                                             use_tc_tiling_on_sc=False),
        scratch_types=[
            pltpu.VMEM((N_PAD,), jnp.float32),
            pltpu.VMEM((RPT,), jnp.int32),
            pltpu.VMEM((CH,), jnp.float32),
            pltpu.VMEM((RPT,), jnp.float32),
            pltpu.VMEM((CH, HALF), jnp.float32),
            pltpu.VMEM((CH, HALF), jnp.float32),
            pltpu.VMEM((CH, HALF), jnp.float32),
            pltpu.VMEM((CH, HALF), jnp.float32),
        ] + [pltpu.VMEM((CH,), jnp.int32) for _ in range(16)]
        + [
            pltpu.VMEM_SHARED((N_PAD, HALF), jnp.float32),
            pltpu.VMEM_SHARED((N_PAD, HALF), jnp.float32),
            pltpu.VMEM_SHARED((N_PAD,), jnp.float32),
        ] + [pltpu.SemaphoreType.DMA] * 13,
    )(row_t, col_t, xflat)


def kernel(edge_index_drop, edge_index, features, preference):
    del edge_index_drop
    x = _tc_normalize(features.astype(jnp.float32))

    xpad = jnp.pad(x, ((0, N_PAD - N_NODES), (0, 0)))
    xflat = xpad.reshape(2 * N_PAD, HALF)

    ei = edge_index.astype(jnp.int32)
    rowp = jnp.pad(ei[0], (0, E_PAD - N_EDGES)).reshape(16, NCH, CH)
    colp = jnp.pad(ei[1], (0, E_PAD - N_EDGES)).reshape(16, NCH, CH)

    out_split, _h = _sc_gcn(rowp, colp, xflat)
    x_hat = out_split.transpose(1, 0, 2).reshape(N_PAD, D_FEAT)[:N_NODES]
    return (x_hat, preference)
